# baseline (speedup 1.0000x reference)
import functools

import jax
import jax.numpy as jnp
from jax.experimental import pallas as pl
from jax.experimental.pallas import tpu as pltpu
from jax.sharding import Mesh, PartitionSpec as P

N = 4096
F = 128
NL = N // 2
HALFL = NL // 2
BKA = 256
JB = HALFL // BKA
LAYERS = 3


def _gcn_kernel(adja_ref, adjb_ref, x_ref, w_ref, b_ref, out_ref,
                abf, h, dinv, sem_sd, sem_rd, sem_sh, sem_rh):
    t = pl.program_id(0)
    my = jax.lax.axis_index("d")
    peer = 1 - my
    my_base = my * NL

    @pl.when(t < JB)
    def _phase_a():
        for base, ref in ((0, adja_ref), (HALFL, adjb_ref)):
            blk = ref[...]
            deg = jnp.sum(blk, axis=1, keepdims=True) + 1.0
            dv = jax.lax.rsqrt(deg + 1e-12)
            dinv[pl.ds(my_base + base + t * BKA, BKA), :] = dv
            abf[pl.ds(base + t * BKA, BKA), :] = blk.astype(jnp.bfloat16)

        @pl.when(t == 0)
        def _():
            h[...] = x_ref[...]

        @pl.when(t == JB - 1)
        def _():
            rdma = pltpu.make_async_remote_copy(
                src_ref=dinv.at[pl.ds(my_base, NL), :],
                dst_ref=dinv.at[pl.ds(my_base, NL), :],
                send_sem=sem_sd, recv_sem=sem_rd,
                device_id=peer,
                device_id_type=pltpu.DeviceIdType.LOGICAL)
            rdma.start()
            rdma.wait()

    @pl.when(t >= JB)
    def _phase_b():
        layer = t - JB
        sup = jnp.dot(h[...], w_ref[layer],
                      preferred_element_type=jnp.float32)
        sup = sup * dinv[...]
        acc = jnp.dot(abf[...], sup.astype(jnp.bfloat16),
                      preferred_element_type=jnp.float32)
        dloc = dinv[pl.ds(my_base, NL), :]
        sup_loc = jnp.dot(h[pl.ds(my_base, NL), :], w_ref[layer],
                          preferred_element_type=jnp.float32) * dloc
        acc = acc + sup_loc
        res = jnp.maximum(acc * dloc + b_ref[layer], 0.0)

        @pl.when(layer < LAYERS - 1)
        def _():
            h[pl.ds(my_base, NL), :] = res
            rdma = pltpu.make_async_remote_copy(
                src_ref=h.at[pl.ds(my_base, NL), :],
                dst_ref=h.at[pl.ds(my_base, NL), :],
                send_sem=sem_sh, recv_sem=sem_rh,
                device_id=peer,
                device_id_type=pltpu.DeviceIdType.LOGICAL)
            rdma.start()
            rdma.wait()

        @pl.when(layer == LAYERS - 1)
        def _():
            out_ref[...] = res


def _per_core(adj_local, x, w, b):
    grid = (JB + LAYERS,)
    return pl.pallas_call(
        _gcn_kernel,
        grid=grid,
        in_specs=[
            pl.BlockSpec((BKA, N), lambda t: (jnp.minimum(t, JB - 1), 0)),
            pl.BlockSpec((BKA, N),
                         lambda t: (JB + jnp.minimum(t, JB - 1), 0)),
            pl.BlockSpec((N, F), lambda t: (0, 0)),
            pl.BlockSpec((LAYERS, F, F), lambda t: (0, 0, 0)),
            pl.BlockSpec((LAYERS, 1, F), lambda t: (0, 0, 0)),
        ],
        out_specs=pl.BlockSpec((NL, F), lambda t: (0, 0)),
        out_shape=jax.ShapeDtypeStruct((NL, F), jnp.float32),
        scratch_shapes=[
            pltpu.VMEM((NL, N), jnp.bfloat16),
            pltpu.VMEM((N, F), jnp.float32),
            pltpu.VMEM((N, 1), jnp.float32),
            pltpu.SemaphoreType.DMA,
            pltpu.SemaphoreType.DMA,
            pltpu.SemaphoreType.DMA,
            pltpu.SemaphoreType.DMA,
        ],
    )(adj_local, adj_local, x, w, b)


def kernel(x, adj, W0, b0, W1, b1, W2, b2):
    w = jnp.stack([W0, W1, W2])
    b = jnp.stack([b0, b1, b2])[:, None, :]
    mesh = Mesh(jax.devices()[:2], ("d",))
    f = jax.shard_map(
        _per_core, mesh=mesh,
        in_specs=(P("d", None), P(None, None), P(None, None, None),
                  P(None, None, None)),
        out_specs=P("d", None),
        check_vma=False,
    )
    return f(adj, x, w, b)


# speedup vs baseline: 7.0259x; 7.0259x over previous
import jax
import jax.numpy as jnp
from jax.experimental import pallas as pl
from jax.experimental.pallas import tpu as pltpu

N = 4096
F = 128
HALF = N // 2
BKA = 256
JB = HALF // BKA
BM = 512
IB = N // BM
LAYERS = 3


def _gcn_kernel(adja_ref, adjb_ref, x_ref, w_ref, b_ref, out_ref,
                abf, h, s16, dinv):
    t = pl.program_id(0)

    @pl.when(t < JB)
    def _phase_a():
        for base, ref in ((0, adja_ref), (HALF, adjb_ref)):
            blk = ref[...]
            deg = jnp.sum(blk, axis=1, keepdims=True) + 1.0
            dv = jax.lax.rsqrt(deg + 1e-12)
            dinv[pl.ds(base + t * BKA, BKA), :] = dv
            abf[pl.ds(base + t * BKA, BKA), :] = blk.astype(jnp.bfloat16)

        @pl.when(t == 0)
        def _():
            h[...] = x_ref[...]

    @pl.when(t >= JB)
    def _phase_b():
        u = t - JB
        layer = u // IB
        i = u % IB

        @pl.when(i == 0)
        def _support():
            sup = jnp.dot(h[...], w_ref[layer],
                          preferred_element_type=jnp.float32)
            sup = sup * dinv[...]
            s16[...] = sup.astype(jnp.bfloat16)

        acc = jnp.dot(abf[pl.ds(i * BM, BM), :], s16[...],
                      preferred_element_type=jnp.float32)
        acc = acc + s16[pl.ds(i * BM, BM), :].astype(jnp.float32)
        res = acc * dinv[pl.ds(i * BM, BM), :] + b_ref[layer]
        res = jnp.maximum(res, 0.0)

        @pl.when(layer < LAYERS - 1)
        def _():
            h[pl.ds(i * BM, BM), :] = res

        @pl.when(layer == LAYERS - 1)
        def _():
            out_ref[pl.ds(i * BM, BM), :] = res


def kernel(x, adj, W0, b0, W1, b1, W2, b2):
    w = jnp.stack([W0, W1, W2])
    b = jnp.stack([b0, b1, b2])[:, None, :]
    grid = (JB + LAYERS * IB,)
    return pl.pallas_call(
        _gcn_kernel,
        grid=grid,
        in_specs=[
            pl.BlockSpec((BKA, N), lambda t: (jnp.minimum(t, JB - 1), 0)),
            pl.BlockSpec((BKA, N),
                         lambda t: (JB + jnp.minimum(t, JB - 1), 0)),
            pl.BlockSpec((N, F), lambda t: (0, 0)),
            pl.BlockSpec((LAYERS, F, F), lambda t: (0, 0, 0)),
            pl.BlockSpec((LAYERS, 1, F), lambda t: (0, 0, 0)),
        ],
        out_specs=pl.BlockSpec((N, F), lambda t: (0, 0)),
        out_shape=jax.ShapeDtypeStruct((N, F), jnp.float32),
        scratch_shapes=[
            pltpu.VMEM((N, N), jnp.bfloat16),
            pltpu.VMEM((N, F), jnp.float32),
            pltpu.VMEM((N, F), jnp.bfloat16),
            pltpu.VMEM((N, 1), jnp.float32),
        ],
    )(adj, adj, x, w, b)


# speedup vs baseline: 7.2212x; 1.0278x over previous
import jax
import jax.numpy as jnp
from jax.experimental import pallas as pl
from jax.experimental.pallas import tpu as pltpu

N = 4096
F = 128
HALF = N // 2
BKA = 256
JB = HALF // BKA
BM = 1024
IB = N // BM
LAYERS = 3


def _gcn_kernel(adja_ref, adjb_ref, x_ref, w_ref, b_ref, out_ref,
                abf, h, s16, dinv):
    t = pl.program_id(0)

    @pl.when(t < JB)
    def _phase_a():
        ones = jnp.ones((N, 8), dtype=jnp.bfloat16)
        for base, ref in ((0, adja_ref), (HALF, adjb_ref)):
            blk = ref[...].astype(jnp.bfloat16)
            abf[pl.ds(base + t * BKA, BKA), :] = blk
            deg = jnp.dot(blk, ones,
                          preferred_element_type=jnp.float32)[:, :1] + 1.0
            dinv[pl.ds(base + t * BKA, BKA), :] = jax.lax.rsqrt(deg + 1e-12)

        @pl.when(t == 0)
        def _():
            h[...] = x_ref[...]

    @pl.when(t >= JB)
    def _phase_b():
        u = t - JB
        layer = u // IB
        i = u % IB

        @pl.when(i == 0)
        def _support():
            sup = jnp.dot(h[...], w_ref[layer],
                          preferred_element_type=jnp.float32)
            sup = sup * dinv[...]
            s16[...] = sup.astype(jnp.bfloat16)

        acc = jnp.dot(abf[pl.ds(i * BM, BM), :], s16[...],
                      preferred_element_type=jnp.float32)
        acc = acc + s16[pl.ds(i * BM, BM), :].astype(jnp.float32)
        res = acc * dinv[pl.ds(i * BM, BM), :] + b_ref[layer]
        res = jnp.maximum(res, 0.0)

        @pl.when(layer < LAYERS - 1)
        def _():
            h[pl.ds(i * BM, BM), :] = res

        @pl.when(layer == LAYERS - 1)
        def _():
            out_ref[pl.ds(i * BM, BM), :] = res


def kernel(x, adj, W0, b0, W1, b1, W2, b2):
    w = jnp.stack([W0, W1, W2])
    b = jnp.stack([b0, b1, b2])[:, None, :]
    grid = (JB + LAYERS * IB,)
    return pl.pallas_call(
        _gcn_kernel,
        grid=grid,
        in_specs=[
            pl.BlockSpec((BKA, N), lambda t: (jnp.minimum(t, JB - 1), 0)),
            pl.BlockSpec((BKA, N),
                         lambda t: (JB + jnp.minimum(t, JB - 1), 0)),
            pl.BlockSpec((N, F), lambda t: (0, 0)),
            pl.BlockSpec((LAYERS, F, F), lambda t: (0, 0, 0)),
            pl.BlockSpec((LAYERS, 1, F), lambda t: (0, 0, 0)),
        ],
        out_specs=pl.BlockSpec((N, F), lambda t: (0, 0)),
        out_shape=jax.ShapeDtypeStruct((N, F), jnp.float32),
        scratch_shapes=[
            pltpu.VMEM((N, N), jnp.bfloat16),
            pltpu.VMEM((N, F), jnp.float32),
            pltpu.VMEM((N, F), jnp.bfloat16),
            pltpu.VMEM((N, 1), jnp.float32),
        ],
    )(adj, adj, x, w, b)


# speedup vs baseline: 7.4134x; 1.0266x over previous
import jax
import jax.numpy as jnp
from jax.experimental import pallas as pl
from jax.experimental.pallas import tpu as pltpu

N = 4096
F = 128
HALF = N // 2
BKA = 256
JB = HALF // BKA
BM = 1024
IB = N // BM
LAYERS = 3


def _gcn_kernel(adja_ref, adjb_ref, x_ref, w_ref, b_ref, out_ref,
                abf, h, s16, dinv):
    t = pl.program_id(0)

    @pl.when(t < JB)
    def _phase_a():
        for base, ref in ((0, adja_ref), (HALF, adjb_ref)):
            blk = ref[...]
            deg = jnp.sum(blk, axis=1, keepdims=True) + 1.0
            dv = jax.lax.rsqrt(deg + 1e-12)
            dinv[pl.ds(base + t * BKA, BKA), :] = dv
            abf[pl.ds(base + t * BKA, BKA), :] = blk.astype(jnp.bfloat16)

        @pl.when(t == 0)
        def _():
            h[...] = x_ref[...]

    @pl.when(t >= JB)
    def _phase_b():
        u = t - JB
        layer = u // IB
        i = u % IB

        @pl.when(i == 0)
        def _support():
            sup = jnp.dot(h[...], w_ref[layer],
                          preferred_element_type=jnp.float32)
            sup = sup * dinv[...]
            s16[...] = sup.astype(jnp.bfloat16)

        acc = jnp.dot(abf[pl.ds(i * BM, BM), :], s16[...],
                      preferred_element_type=jnp.float32)
        acc = acc + s16[pl.ds(i * BM, BM), :].astype(jnp.float32)
        res = acc * dinv[pl.ds(i * BM, BM), :] + b_ref[layer]
        res = jnp.maximum(res, 0.0)

        @pl.when(layer < LAYERS - 1)
        def _():
            h[pl.ds(i * BM, BM), :] = res

        @pl.when(layer == LAYERS - 1)
        def _():
            out_ref[pl.ds(i * BM, BM), :] = res


def kernel(x, adj, W0, b0, W1, b1, W2, b2):
    w = jnp.stack([W0, W1, W2])
    b = jnp.stack([b0, b1, b2])[:, None, :]
    grid = (JB + LAYERS * IB,)
    return pl.pallas_call(
        _gcn_kernel,
        grid=grid,
        in_specs=[
            pl.BlockSpec((BKA, N), lambda t: (jnp.minimum(t, JB - 1), 0)),
            pl.BlockSpec((BKA, N),
                         lambda t: (JB + jnp.minimum(t, JB - 1), 0)),
            pl.BlockSpec((N, F), lambda t: (0, 0)),
            pl.BlockSpec((LAYERS, F, F), lambda t: (0, 0, 0)),
            pl.BlockSpec((LAYERS, 1, F), lambda t: (0, 0, 0)),
        ],
        out_specs=pl.BlockSpec((N, F), lambda t: (0, 0)),
        out_shape=jax.ShapeDtypeStruct((N, F), jnp.float32),
        scratch_shapes=[
            pltpu.VMEM((N, N), jnp.bfloat16),
            pltpu.VMEM((N, F), jnp.float32),
            pltpu.VMEM((N, F), jnp.bfloat16),
            pltpu.VMEM((N, 1), jnp.float32),
        ],
    )(adj, adj, x, w, b)
